# baseline (speedup 1.0000x reference)
import functools

import jax
import jax.numpy as jnp
from jax import lax
from jax.experimental import pallas as pl
from jax.experimental.pallas import tpu as pltpu
from jax.experimental.pallas import tpu_sc as plsc

_R = 8
_BN = 512


def _pool_sc(x4, n, c, s):
    info = plsc.get_sparse_core_info()
    nw = info.num_cores * info.num_subcores
    nchunks = n // _R
    iters = (nchunks + nw - 1) // nw
    mesh = plsc.VectorSubcoreMesh(core_axis_name="c", subcore_axis_name="s")
    vecs = (_R * c) // 16

    @functools.partial(
        pl.kernel,
        mesh=mesh,
        out_type=jax.ShapeDtypeStruct((n, c), jnp.float32),
        scratch_types=[
            pltpu.VMEM((s, _R, c), jnp.float32),
            pltpu.VMEM((_R, c), jnp.float32),
            pltpu.SemaphoreType.DMA,
            pltpu.SemaphoreType.DMA,
        ],
    )
    def pool(x_hbm, out_hbm, slab_v, outst_v, sem_in, sem_out):
        wid = lax.axis_index("s") * info.num_cores + lax.axis_index("c")

        def chunk_body(ci, _):
            chunk = ci * nw + wid

            @pl.when(chunk < nchunks)
            def _():
                row0 = chunk * _R
                copies = [
                    pltpu.async_copy(
                        x_hbm.at[si, pl.ds(row0, _R), :], slab_v.at[si], sem_in
                    )
                    for si in range(s)
                ]
                for cp in copies:
                    cp.wait()

                def vec_body(v, _):
                    r = v // (c // 16)
                    k = (v % (c // 16)) * 16
                    acc = slab_v[0, r, pl.ds(k, 16)]
                    for si in range(1, s):
                        acc = acc + slab_v[si, r, pl.ds(k, 16)]
                    outst_v[r, pl.ds(k, 16)] = acc * (1.0 / s)
                    return 0

                lax.fori_loop(0, vecs, vec_body, 0)
                pltpu.async_copy(
                    outst_v, out_hbm.at[pl.ds(row0, _R), :], sem_out
                ).wait()

            return 0

        lax.fori_loop(0, iters, chunk_body, 0)

    return pool(x4)


def _heads_body(xm_ref, wc_ref, bc_ref, wr_ref, br_ref, cls_ref, reg_ref):
    xm = xm_ref[...]
    dn = (((1,), (1,)), ((), ()))
    cls_ref[...] = (
        lax.dot_general(wc_ref[...], xm, dn, preferred_element_type=jnp.float32)
        + bc_ref[...]
    )
    reg_ref[...] = (
        lax.dot_general(wr_ref[...], xm, dn, preferred_element_type=jnp.float32)
        + br_ref[...]
    )


def kernel(x, W_cls, b_cls, W_reg, b_reg):
    n, c, rh, rw = x.shape
    s = rh * rw
    k1 = W_cls.shape[0]
    k2 = W_reg.shape[0]
    x4 = x.transpose(2, 3, 0, 1).reshape(s, n, c)
    xm = _pool_sc(x4, n, c, s)
    bc2 = b_cls.reshape(k1, 1)
    br2 = b_reg.reshape(k2, 1)
    grid = (n + _BN - 1) // _BN
    cls_t, reg_t = pl.pallas_call(
        _heads_body,
        grid=(grid,),
        in_specs=[
            pl.BlockSpec((_BN, c), lambda i: (i, 0)),
            pl.BlockSpec((k1, c), lambda i: (0, 0)),
            pl.BlockSpec((k1, 1), lambda i: (0, 0)),
            pl.BlockSpec((k2, c), lambda i: (0, 0)),
            pl.BlockSpec((k2, 1), lambda i: (0, 0)),
        ],
        out_specs=[
            pl.BlockSpec((k1, _BN), lambda i: (0, i)),
            pl.BlockSpec((k2, _BN), lambda i: (0, i)),
        ],
        out_shape=[
            jax.ShapeDtypeStruct((k1, n), jnp.float32),
            jax.ShapeDtypeStruct((k2, n), jnp.float32),
        ],
    )(xm, W_cls, bc2, W_reg, br2)
    return (cls_t.T, reg_t.T)


# speedup vs baseline: 2.0032x; 2.0032x over previous
import functools

import jax
import jax.numpy as jnp
from jax import lax
from jax.experimental import pallas as pl
from jax.experimental.pallas import tpu as pltpu
from jax.experimental.pallas import tpu_sc as plsc

_R = 8
_BN = 128
_BH = 512
_N_SC = 6144


def _pool_sc(x4, n_sc, c, s):
    info = plsc.get_sparse_core_info()
    nw = info.num_cores * info.num_subcores
    nchunks = n_sc // _R
    iters = (nchunks + nw - 1) // nw
    mesh = plsc.VectorSubcoreMesh(core_axis_name="c", subcore_axis_name="s")
    vecs = (_R * c) // 16

    @functools.partial(
        pl.kernel,
        mesh=mesh,
        out_type=jax.ShapeDtypeStruct((n_sc, c), jnp.float32),
        scratch_types=[
            pltpu.VMEM((s, _R, c), jnp.float32),
            pltpu.VMEM((_R, c), jnp.float32),
            pltpu.SemaphoreType.DMA,
            pltpu.SemaphoreType.DMA,
        ],
    )
    def pool(x_hbm, out_hbm, slab_v, outst_v, sem_in, sem_out):
        wid = lax.axis_index("s") * info.num_cores + lax.axis_index("c")

        def chunk_body(ci, _):
            chunk = ci * nw + wid

            @pl.when(chunk < nchunks)
            def _():
                row0 = chunk * _R
                copies = [
                    pltpu.async_copy(
                        x_hbm.at[si, pl.ds(row0, _R), :], slab_v.at[si], sem_in
                    )
                    for si in range(s)
                ]
                for cp in copies:
                    cp.wait()

                def vec_body(v, _):
                    r = v // (c // 16)
                    k = (v % (c // 16)) * 16
                    acc = slab_v[0, r, pl.ds(k, 16)]
                    for si in range(1, s):
                        acc = acc + slab_v[si, r, pl.ds(k, 16)]
                    outst_v[r, pl.ds(k, 16)] = acc * (1.0 / s)
                    return 0

                lax.fori_loop(0, vecs, vec_body, 0)
                pltpu.async_copy(
                    outst_v, out_hbm.at[pl.ds(row0, _R), :], sem_out
                ).wait()

            return 0

        lax.fori_loop(0, iters, chunk_body, 0)

    return pool(x4)


def _main_body(x_ref, wc_ref, bc_ref, wr_ref, br_ref, cls_ref, reg_ref):
    s = x_ref.shape[0]
    xm = jnp.sum(x_ref[...], axis=0) * (1.0 / s)
    dn = (((1,), (1,)), ((), ()))
    cls_ref[...] = (
        lax.dot_general(wc_ref[...], xm, dn, preferred_element_type=jnp.float32)
        + bc_ref[...]
    )
    reg_ref[...] = (
        lax.dot_general(wr_ref[...], xm, dn, preferred_element_type=jnp.float32)
        + br_ref[...]
    )


def _heads_body(xm_ref, wc_ref, bc_ref, wr_ref, br_ref, cls_ref, reg_ref):
    xm = xm_ref[...]
    dn = (((1,), (1,)), ((), ()))
    cls_ref[...] = (
        lax.dot_general(wc_ref[...], xm, dn, preferred_element_type=jnp.float32)
        + bc_ref[...]
    )
    reg_ref[...] = (
        lax.dot_general(wr_ref[...], xm, dn, preferred_element_type=jnp.float32)
        + br_ref[...]
    )


def kernel(x, W_cls, b_cls, W_reg, b_reg):
    n, c, rh, rw = x.shape
    s = rh * rw
    k1 = W_cls.shape[0]
    k2 = W_reg.shape[0]
    x4 = x.transpose(2, 3, 0, 1).reshape(s, n, c)
    bc2 = b_cls.reshape(k1, 1)
    br2 = b_reg.reshape(k2, 1)

    xm_sc = _pool_sc(x4, _N_SC, c, s)

    off = _N_SC // _BN
    grid = (n - _N_SC + _BN - 1) // _BN
    cls_t, reg_t = pl.pallas_call(
        _main_body,
        grid=(grid,),
        in_specs=[
            pl.BlockSpec((s, _BN, c), lambda i: (0, i + off, 0)),
            pl.BlockSpec((k1, c), lambda i: (0, 0)),
            pl.BlockSpec((k1, 1), lambda i: (0, 0)),
            pl.BlockSpec((k2, c), lambda i: (0, 0)),
            pl.BlockSpec((k2, 1), lambda i: (0, 0)),
        ],
        out_specs=[
            pl.BlockSpec((k1, _BN), lambda i: (0, i + off)),
            pl.BlockSpec((k2, _BN), lambda i: (0, i + off)),
        ],
        out_shape=[
            jax.ShapeDtypeStruct((k1, n), jnp.float32),
            jax.ShapeDtypeStruct((k2, n), jnp.float32),
        ],
    )(x4, W_cls, bc2, W_reg, br2)

    cls_sc, reg_sc = pl.pallas_call(
        _heads_body,
        grid=(_N_SC // _BH,),
        in_specs=[
            pl.BlockSpec((_BH, c), lambda i: (i, 0)),
            pl.BlockSpec((k1, c), lambda i: (0, 0)),
            pl.BlockSpec((k1, 1), lambda i: (0, 0)),
            pl.BlockSpec((k2, c), lambda i: (0, 0)),
            pl.BlockSpec((k2, 1), lambda i: (0, 0)),
        ],
        out_specs=[
            pl.BlockSpec((k1, _BH), lambda i: (0, i)),
            pl.BlockSpec((k2, _BH), lambda i: (0, i)),
        ],
        out_shape=[
            jax.ShapeDtypeStruct((k1, _N_SC), jnp.float32),
            jax.ShapeDtypeStruct((k2, _N_SC), jnp.float32),
        ],
    )(xm_sc, W_cls, bc2, W_reg, br2)

    cls_t = lax.dynamic_update_slice(cls_t, cls_sc, (0, 0))
    reg_t = lax.dynamic_update_slice(reg_t, reg_sc, (0, 0))
    return (cls_t.T, reg_t.T)
